# Initial kernel scaffold; baseline (speedup 1.0000x reference)
import jax
import jax.numpy as jnp
from jax.experimental import pallas as pl


def kernel(x):
    raise NotImplementedError("write your pallas kernel here")


# speedup vs baseline: 2.3619x; 2.3619x over previous
import jax
import jax.numpy as jnp
from jax.experimental import pallas as pl
from jax.experimental.pallas import tpu as pltpu

_B = 16
_C = 512
_HW = 1024
_ROTS = ((13, 15, 26, 6), (17, 29, 16, 24))
_KS = (0, 42, (0 ^ 42 ^ 0x1BD11BDA) & 0xFFFFFFFF)


def _threefry_bits(j):
    x0 = jnp.zeros_like(j)
    x1 = j + jnp.uint32(_KS[1])
    for i in range(5):
        for r in _ROTS[i % 2]:
            x0 = x0 + x1
            x1 = (x1 << jnp.uint32(r)) | (x1 >> jnp.uint32(32 - r))
            x1 = x1 ^ x0
        x0 = x0 + jnp.uint32(_KS[(i + 1) % 3])
        x1 = x1 + jnp.uint32((_KS[(i + 2) % 3] + i + 1) & 0xFFFFFFFF)
    return x0 ^ x1


def _body(x_ref, zq_ref, ind_ref, perp_ref, acc_ref):
    b = pl.program_id(0)

    c_iota = jax.lax.broadcasted_iota(jnp.uint32, (_C, _HW), 0)
    hw_iota = jax.lax.broadcasted_iota(jnp.uint32, (_C, _HW), 1)
    j = (b * (_HW * _C)).astype(jnp.uint32) + hw_iota * jnp.uint32(_C) + c_iota
    bits = _threefry_bits(j)
    fbits = (bits >> jnp.uint32(9)) | jnp.uint32(0x3F800000)
    u = jax.lax.bitcast_convert_type(fbits, jnp.float32) - jnp.float32(1.0)
    g = -jnp.log(-jnp.log(u + 1e-20) + 1e-20)

    s = x_ref[0] + g

    ci32 = jax.lax.broadcasted_iota(jnp.int32, (_C, _HW), 0)
    m = jnp.max(s, axis=0, keepdims=True)
    ind = jnp.min(jnp.where(s == m, ci32, _C), axis=0, keepdims=True)
    oh = (ci32 == ind).astype(jnp.float32)
    zq_ref[0] = oh
    ind_ref[0] = ind

    partial = sum(oh[:, k * 128:(k + 1) * 128] for k in range(_HW // 128))

    @pl.when(b == 0)
    def _():
        acc_ref[...] = partial

    @pl.when(b != 0)
    def _():
        acc_ref[...] = acc_ref[...] + partial

    @pl.when(b == _B - 1)
    def _():
        counts = jnp.sum(acc_ref[...], axis=1, keepdims=True)
        p = counts * jnp.float32(1.0 / (_B * _HW))
        ent = jnp.sum(p * jnp.log(p + 1e-10), keepdims=True)
        perp_ref[...] = jnp.exp(-ent)


def _quantize(x3):
    return pl.pallas_call(
        _body,
        grid=(_B,),
        in_specs=[pl.BlockSpec((1, _C, _HW), lambda b: (b, 0, 0))],
        out_specs=[
            pl.BlockSpec((1, _C, _HW), lambda b: (b, 0, 0)),
            pl.BlockSpec((1, 1, _HW), lambda b: (b, 0, 0)),
            pl.BlockSpec((1, 1), lambda b: (0, 0)),
        ],
        out_shape=[
            jax.ShapeDtypeStruct((_B, _C, _HW), jnp.float32),
            jax.ShapeDtypeStruct((_B, 1, _HW), jnp.int32),
            jax.ShapeDtypeStruct((1, 1), jnp.float32),
        ],
        scratch_shapes=[pltpu.VMEM((_C, 128), jnp.float32)],
        compiler_params=pltpu.CompilerParams(
            dimension_semantics=("arbitrary",),
        ),
    )(x3)


def kernel(x):
    b, c, h, w = x.shape
    x3 = x.reshape(b, c, h * w)
    zq, ind, perp = _quantize(x3)
    return (
        zq.reshape(b, c, h, w),
        0.0,
        ind.reshape(b, h, w),
        perp[0, 0],
    )
